# baseline (speedup 1.0000x reference)
import functools

import jax
import jax.numpy as jnp
from jax import lax
from jax.experimental import pallas as pl
from jax.experimental.pallas import tpu as pltpu
from jax.experimental.pallas import tpu_sc as plsc

NC = 2
NS = 16
NW = NC * NS

D = 2048
FF = 1024
E = 64
TOPK = 8
T = 2048
BT = 256
NB = T * TOPK // BT + E
NROWS = NB * BT


def _router_body(hs_ref, wr_ref, w_ref, i_ref):
    hs = hs_ref[...]
    logits = jnp.dot(hs, wr_ref[...], preferred_element_type=jnp.float32)
    m0 = jnp.max(logits, axis=1, keepdims=True)
    denom = jnp.sum(jnp.exp(logits - m0), axis=1, keepdims=True)
    cols = lax.broadcasted_iota(jnp.int32, (T, E), 1)
    cols_out = lax.broadcasted_iota(jnp.int32, (T, 128), 1)
    work = logits
    w_acc = jnp.zeros((T, 128), jnp.float32)
    i_acc = jnp.zeros((T, 128), jnp.int32)
    for k in range(TOPK):
        m = jnp.max(work, axis=1, keepdims=True)
        amax = jnp.min(jnp.where(work == m, cols, E), axis=1, keepdims=True)
        prob = jnp.exp(m - m0) / denom
        sel = (cols_out == k).astype(jnp.float32)
        w_acc = w_acc + prob * sel
        i_acc = i_acc + amax * sel.astype(jnp.int32)
        work = jnp.where(cols == amax, -jnp.inf, work)
    w_ref[...] = w_acc
    i_ref[...] = i_acc


def _router(hs, router_weight):
    return pl.pallas_call(
        _router_body,
        out_shape=(
            jax.ShapeDtypeStruct((T, 128), jnp.float32),
            jax.ShapeDtypeStruct((T, 128), jnp.int32),
        ),
    )(hs, router_weight.T)


def _gmm_body(be_ref, x_ref, gu_ref, dp_ref, y_ref):
    b = pl.program_id(0)
    nb = be_ref[NB]

    @pl.when(b < nb)
    def _():
        x = x_ref[...].astype(jnp.bfloat16)
        gu = gu_ref[0].astype(jnp.bfloat16)
        h = jnp.dot(x, gu, preferred_element_type=jnp.float32)
        gate = h[:, :FF]
        up = h[:, FF:]
        act = (gate * lax.logistic(gate) * up).astype(jnp.bfloat16)
        dp = dp_ref[0].astype(jnp.bfloat16)
        y_ref[...] = jnp.dot(act, dp, preferred_element_type=jnp.float32)


def _gmm(be_arr, x_sorted, gu_bf16, dp_bf16):
    grid_spec = pltpu.PrefetchScalarGridSpec(
        num_scalar_prefetch=1,
        grid=(NB,),
        in_specs=[
            pl.BlockSpec((BT, D), lambda b, be: (b, 0)),
            pl.BlockSpec((1, D, 2 * FF), lambda b, be: (be[b], 0, 0)),
            pl.BlockSpec((1, FF, D), lambda b, be: (be[b], 0, 0)),
        ],
        out_specs=pl.BlockSpec((BT, D), lambda b, be: (b, 0)),
    )
    return pl.pallas_call(
        _gmm_body,
        grid_spec=grid_spec,
        out_shape=jax.ShapeDtypeStruct((NROWS, D), jnp.float32),
    )(be_arr, x_sorted, gu_bf16, dp_bf16)


_GCH = 32
_RPW = NROWS // NW
_TPW = T // NW
_CT = 4


def _sc_gather_body(hs_hbm, tok_hbm, x_hbm, idx_v, rows_v, sem):
    wid = lax.axis_index("s") * NC + lax.axis_index("c")
    base = wid * _RPW
    pltpu.sync_copy(tok_hbm.at[pl.ds(base, _RPW)], idx_v)

    def chunk(c, carry):
        cb = c * _GCH
        pltpu.async_copy(hs_hbm.at[idx_v.at[pl.ds(cb, _GCH)]], rows_v, sem).wait()
        pltpu.sync_copy(rows_v, x_hbm.at[pl.ds(base + cb, _GCH)])
        return carry

    lax.fori_loop(0, _RPW // _GCH, chunk, 0)


def _sc_gather(hs_b, sorted_tok):
    mesh = plsc.VectorSubcoreMesh(core_axis_name="c", subcore_axis_name="s")
    f = functools.partial(
        pl.kernel, mesh=mesh,
        out_type=jax.ShapeDtypeStruct((NROWS, D), jnp.float32),
        scratch_types=[
            pltpu.VMEM((_RPW,), jnp.int32),
            pltpu.VMEM((_GCH, D), jnp.float32),
            pltpu.SemaphoreType.DMA,
        ],
    )(_sc_gather_body)
    return f(hs_b, sorted_tok)


def _sc_combine_body(y_hbm, pos_hbm, w_hbm, out_hbm, idx_v, w_v, rows_v, out_v, sem):
    wid = lax.axis_index("s") * NC + lax.axis_index("c")
    tok_base = wid * _TPW
    pbase = tok_base * TOPK
    pltpu.sync_copy(pos_hbm.at[pl.ds(pbase, _TPW * TOPK)], idx_v)
    pltpu.sync_copy(w_hbm.at[pl.ds(pbase, _TPW * TOPK)], w_v)

    def chunk(c, carry):
        cb = c * _CT * TOPK
        pltpu.async_copy(y_hbm.at[idx_v.at[pl.ds(cb, _CT * TOPK)]], rows_v, sem).wait()
        for tt in range(_CT):
            w16 = w_v[pl.ds(cb + tt * TOPK, 16)]
            wvecs = [jnp.full((16,), w16[k], jnp.float32) for k in range(TOPK)]

            def inner(v, carry2):
                sl = pl.ds(v * 16, 16)
                acc = wvecs[0] * rows_v[tt * TOPK + 0, sl]
                for k in range(1, TOPK):
                    acc = acc + wvecs[k] * rows_v[tt * TOPK + k, sl]
                out_v[tt, sl] = acc
                return carry2

            lax.fori_loop(0, D // 16, inner, 0)
        pltpu.sync_copy(out_v, out_hbm.at[pl.ds(tok_base + c * _CT, _CT)])
        return carry

    lax.fori_loop(0, _TPW // _CT, chunk, 0)


def _sc_combine(y, pos_flat, w_flat):
    mesh = plsc.VectorSubcoreMesh(core_axis_name="c", subcore_axis_name="s")
    f = functools.partial(
        pl.kernel, mesh=mesh,
        out_type=jax.ShapeDtypeStruct((T, D), jnp.float32),
        scratch_types=[
            pltpu.VMEM((_TPW * TOPK,), jnp.int32),
            pltpu.VMEM((_TPW * TOPK,), jnp.float32),
            pltpu.VMEM((_CT * TOPK, D), jnp.float32),
            pltpu.VMEM((_CT, D), jnp.float32),
            pltpu.SemaphoreType.DMA,
        ],
    )(_sc_combine_body)
    return f(y, pos_flat, w_flat)


def kernel(hidden_states, router_weight, gate_up_proj, down_proj):
    hs = hidden_states.reshape(T, D)

    w_pad, i_pad = _router(hs, router_weight)
    top_w = w_pad[:, :TOPK]
    top_i = i_pad[:, :TOPK]

    e_flat = top_i.reshape(-1)
    onehot = (e_flat[:, None] == jnp.arange(E, dtype=jnp.int32)[None, :])
    csum = jnp.cumsum(onehot.astype(jnp.int32), axis=0)
    counts = csum[-1]
    rank = jnp.take_along_axis(csum, e_flat[:, None], axis=1)[:, 0] - 1
    blocks = (counts + BT - 1) // BT
    blocks_incl = jnp.cumsum(blocks)
    nb = blocks_incl[-1].astype(jnp.int32)
    blk_start = blocks_incl - blocks
    pad_off = BT * blk_start

    pos_flat = pad_off[e_flat] + rank
    pos = pos_flat.reshape(T, TOPK)
    sorted_tok = jnp.zeros((NROWS,), jnp.int32).at[pos_flat].set(
        (jnp.arange(T * TOPK, dtype=jnp.int32) // TOPK))

    be = jnp.searchsorted(blocks_incl, jnp.arange(NB, dtype=jnp.int32),
                          side="right").astype(jnp.int32)
    be_last = jnp.clip(be, 0, E - 1)
    last = be_last[jnp.maximum(nb - 1, 0)]
    be = jnp.where(jnp.arange(NB) < nb, be_last, last)
    be_arr = jnp.concatenate([be, nb[None]])

    x_sorted = _sc_gather(hs, sorted_tok)
    y = _gmm(be_arr, x_sorted, gate_up_proj, down_proj)
    out = _sc_combine(y, pos_flat, top_w.reshape(-1))
    return out.reshape(1, T, D)
